# baseline (speedup 1.0000x reference)
import math

import numpy as np

import jax
import jax.numpy as jnp
from jax.experimental import pallas as pl
from jax.experimental.pallas import tpu as pltpu

_BN_EPS = 1e-5
_SLOPE = 0.01
_VLIM = 32 * 1024 * 1024
_TAPS9 = [(dy, dw) for dy in range(3) for dw in range(3)]

_LV = [
    (4, 4, 8, 8, 256, 128),
    (8, 8, 16, 16, 128, 64),
    (16, 16, 32, 32, 64, 32),
    (32, 32, 64, 64, 32, 16),
]
_WQ = 8
_WQP = 10
_GQ = 11


def _pk_rows(h):
    rcore = (h + 2) * _WQP
    return rcore, rcore + 2 * _GQ


def _upmat(n_in, n_out):
    u = np.zeros((n_out, n_in), np.float32)
    if n_in == 1:
        u[:, 0] = 1.0
        return u
    for i in range(n_out):
        s = i * (n_in - 1) / (n_out - 1)
        lo = int(math.floor(s))
        hi = min(lo + 1, n_in - 1)
        f = s - lo
        u[i, lo] += 1.0 - f
        u[i, hi] += f
    return u


def _hlerp_taps(h1, h):
    taps = []
    for i in range(h):
        s = i * (h1 - 1) / (h - 1) if h > 1 else 0.0
        lo = int(math.floor(s))
        hi = min(lo + 1, h1 - 1)
        taps.append((lo, hi, s - lo))
    return taps


def _pk_mask(h):
    m = np.zeros((h + 2, _WQP), np.float32)
    m[1:h + 1, 1:_WQ + 1] = 1.0
    return m.reshape(-1, 1)


def _pack_taps(k_hwio, p, n_pad=128):
    k = k_hwio.astype(jnp.float32)
    cin, cout = k.shape[2], k.shape[3]
    m = np.zeros((3, 3, p, p), np.float32)
    for wi, dw in enumerate((-1, 0, 1)):
        for xi, dx in enumerate((-1, 0, 1)):
            for i in range(p):
                for j in range(p):
                    if dw * p + i - j == dx:
                        m[wi, xi, i, j] = 1.0
    t = jnp.einsum('yxco,wxij->ywicjo', k, jnp.asarray(m))
    t = t.reshape(9, p * cin, p * cout)
    if p * cout < n_pad:
        t = jnp.pad(t, ((0, 0), (0, 0), (0, n_pad - p * cout)))
    return t.astype(jnp.bfloat16)


def _tile_row(v, p, n_pad=128):
    v = jnp.tile(v.astype(jnp.float32).reshape(-1), p).reshape(1, -1)
    if v.shape[1] < n_pad:
        v = jnp.pad(v, ((0, 0), (0, n_pad - v.shape[1])))
    return v


def _bn_fold(bias, bn):
    s = bn["gamma"] / jnp.sqrt(bn["var"] + _BN_EPS)
    t = (bias - bn["mean"]) * s + bn["beta"]
    return s, t


def _body(*refs):
    xs = refs[0:5]
    masks = refs[5:9]
    wrefs = refs[9:49]
    wout, bout = refs[49], refs[50]
    o_out, y1o, y2o, y3o = refs[51:55]
    z, up, yt, acc = refs[55:59]
    ybufs = refs[59:63]
    youts = [y1o, y2o, y3o, None]

    for lvl in range(4):
        h1, w1, h, w, c1, c2 = _LV[lvl]
        p2 = 128 // c2
        rcq, rxq = _pk_rows(h)
        if lvl == 0:
            z_wp, z_g, z_rows, z_lanes, slen = 6, 7, 50, 128, w1
        else:
            z_wp, z_g = _WQP, _GQ
            z_rows = _pk_rows(h1)[1]
            z_lanes, slen = 64, _WQ
        (w1pk, b1pk, uw, w1a, w1b, s1, t1, w2c, s2, t2) = \
            wrefs[10 * lvl:10 * lvl + 10]

        if lvl == 0:
            xin = xs[0][0]
        else:
            xin = ybufs[lvl - 1][...].astype(jnp.bfloat16)
        z[pl.ds(0, z_rows), :] = (
            jnp.dot(xin, w1pk[...], preferred_element_type=jnp.float32)
            + b1pk[...])

        p1 = p2 // 2
        if lvl == 0:
            bmat = jnp.concatenate(
                [z[pl.ds(z_g + (r + 1) * z_wp + 1, w1), :z_lanes]
                 for r in range(h1)], axis=1)
        else:
            bmat = jnp.concatenate(
                [jnp.concatenate(
                    [z[pl.ds(z_g + (r + 1) * z_wp + 1, _WQ),
                       k * c2:(k + 1) * c2] for r in range(h1)], axis=1)
                 for k in range(p1)], axis=0)
        res = jnp.dot(uw[...], bmat, preferred_element_type=jnp.float32)
        up[pl.ds(0, rxq), :] = jnp.zeros((rxq, 128), up.dtype)
        for i, (lo, hi, f) in enumerate(_hlerp_taps(h1, h)):
            rl = res[:, lo * c2:(lo + 1) * c2]
            if f == 0.0:
                rowv = rl
            else:
                rowv = (1.0 - f) * rl + f * res[:, hi * c2:(hi + 1) * c2]
            if p2 == 1:
                rp = rowv
            else:
                rp = jnp.concatenate(
                    [rowv[j * _WQ:(j + 1) * _WQ, :] for j in range(p2)], axis=1)
            up[pl.ds(_GQ + (i + 1) * _WQP + 1, _WQ), :] = rp.astype(up.dtype)

        x2r = xs[lvl + 1]
        acc[pl.ds(0, rcq), :] = jnp.zeros((rcq, 128), jnp.float32)
        for t, (dy, dw) in enumerate(_TAPS9):
            off = _GQ + (dy - 1) * _WQP + (dw - 1)
            acc[pl.ds(0, rcq), :] += jnp.dot(
                x2r[0, pl.ds(off, rcq), :], w1a[t],
                preferred_element_type=jnp.float32)
            acc[pl.ds(0, rcq), :] += jnp.dot(
                up[pl.ds(off, rcq), :], w1b[t],
                preferred_element_type=jnp.float32)
        y1v = acc[pl.ds(0, rcq), :] * s1[...] + t1[...]
        y1v = jnp.where(y1v >= 0.0, y1v, _SLOPE * y1v) * masks[lvl][...]
        yt[pl.ds(0, _GQ), :] = jnp.zeros((_GQ, 128), yt.dtype)
        yt[pl.ds(_GQ, rcq), :] = y1v.astype(yt.dtype)
        yt[pl.ds(_GQ + rcq, _GQ), :] = jnp.zeros((_GQ, 128), yt.dtype)

        acc[pl.ds(0, rcq), :] = jnp.zeros((rcq, 128), jnp.float32)
        for t, (dy, dw) in enumerate(_TAPS9):
            off = _GQ + (dy - 1) * _WQP + (dw - 1)
            acc[pl.ds(0, rcq), :] += jnp.dot(
                yt[pl.ds(off, rcq), :], w2c[t],
                preferred_element_type=jnp.float32)
        y2v = acc[pl.ds(0, rcq), :] * s2[...] + t2[...]
        y2v = jnp.where(y2v >= 0.0, y2v, _SLOPE * y2v) * masks[lvl][...]

        yb = ybufs[lvl]
        yb[pl.ds(0, _GQ), :] = jnp.zeros((_GQ, 128), yb.dtype)
        yb[pl.ds(_GQ, rcq), :] = y2v.astype(yb.dtype)
        yb[pl.ds(_GQ + rcq, _GQ), :] = jnp.zeros((_GQ, 128), yb.dtype)
        if lvl < 3:
            youts[lvl][0] = y2v

    rcq = _pk_rows(64)[0]
    acc[pl.ds(0, rcq), :] = jnp.zeros((rcq, 128), jnp.float32)
    for t, (dy, dw) in enumerate(_TAPS9):
        off = _GQ + (dy - 1) * _WQP + (dw - 1)
        acc[pl.ds(0, rcq), :] += jnp.dot(
            ybufs[3][pl.ds(off, rcq), :], wout[t],
            preferred_element_type=jnp.float32)
    o_out[0] = acc[pl.ds(0, rcq), :32] + bout[:, :32]


def _pack_ext(x_nchw, p):
    n, c, h, w = x_nchw.shape
    wq = (w * c) // 128
    x = jnp.transpose(x_nchw, (0, 2, 3, 1)).reshape(n, h, wq, 128)
    xp = jnp.pad(x, ((0, 0), (1, 1), (1, 1), (0, 0)))
    flat = xp.reshape(n, (h + 2) * (wq + 2), 128)
    return jnp.pad(flat, ((0, 0), (_GQ, _GQ), (0, 0))).astype(jnp.bfloat16)


def _ext_unpacked(x_nchw):
    n, c, h, w = x_nchw.shape
    x = jnp.transpose(x_nchw, (0, 2, 3, 1))
    xp = jnp.pad(x, ((0, 0), (1, 1), (1, 1), (0, 0)))
    flat = xp.reshape(n, (h + 2) * (w + 2), c)
    g = w + 3
    return jnp.pad(flat, ((0, 0), (g, g), (0, 0))).astype(jnp.bfloat16)


def _unpack_out(a, h, p, c):
    n = a.shape[0]
    b = a.reshape(n, h + 2, _WQP, a.shape[-1])[:, 1:h + 1, 1:_WQ + 1, :p * c]
    b = b.reshape(n, h, _WQ * p, c)
    return jnp.transpose(b, (0, 3, 1, 2))


def _decoder(feats_nchw, params):
    n = feats_nchw[0].shape[0]
    exts = [_ext_unpacked(feats_nchw[4])]
    for lvl in range(4):
        c2 = _LV[lvl][5]
        exts.append(_pack_ext(feats_nchw[3 - lvl], 128 // c2))
    masks = [jnp.asarray(_pk_mask(lv[2])) for lv in _LV]

    wlist = []
    for lvl, name in enumerate(("up1", "up2", "up3", "up4")):
        p = params[name]
        h1, w1, h, w, c1, c2 = _LV[lvl]
        p2 = 128 // c2
        p1 = p2 // 2
        w1x1 = p["w1x1"].astype(jnp.float32)
        if lvl == 0:
            w1pk = w1x1.astype(jnp.bfloat16)
            b1pk = _tile_row(p["b1x1"], 1)
        else:
            w1pk = jnp.einsum('co,ij->icjo', w1x1, jnp.eye(p1, dtype=jnp.float32))
            w1pk = w1pk.reshape(p1 * c1, p1 * c2)
            w1pk = jnp.pad(w1pk, ((0, 0), (0, 128 - p1 * c2))).astype(jnp.bfloat16)
            b1pk = _tile_row(p["b1x1"], p1)
        u_np = _upmat(w1, w)
        if lvl > 0:
            rmap = np.array([(r % _WQ) * p2 + r // _WQ for r in range(w)])
            cmap = np.array([(c % _WQ) * p1 + c // _WQ for c in range(w1)])
            u_np = u_np[np.ix_(rmap, cmap)]
        uw = jnp.asarray(u_np)
        wc1 = p["conv"]["w1"]
        w1a = _pack_taps(wc1[:, :, :c2, :], p2)
        w1b = _pack_taps(wc1[:, :, c2:, :], p2)
        w2c = _pack_taps(p["conv"]["w2"], p2)
        s1, t1 = _bn_fold(p["conv"]["b1"], p["conv"]["bn1"])
        s2, t2 = _bn_fold(p["conv"]["b2"], p["conv"]["bn2"])
        wlist += [w1pk, b1pk, uw, w1a, w1b, _tile_row(s1, p2), _tile_row(t1, p2),
                  w2c, _tile_row(s2, p2), _tile_row(t2, p2)]

    wout = _pack_taps(params["out_conv"]["w"], 8)
    bout = _tile_row(params["out_conv"]["b"], 8)

    rcores = [_pk_rows(lv[2])[0] for lv in _LV]
    rexts = [_pk_rows(lv[2])[1] for lv in _LV]

    in_specs = []
    ins = []
    for e in exts:
        in_specs.append(pl.BlockSpec((1, e.shape[1], e.shape[2]),
                                     lambda b: (b, 0, 0)))
        ins.append(e)
    for m in masks:
        in_specs.append(pl.BlockSpec(m.shape, lambda b: (0, 0)))
        ins.append(m)
    for wq_ in wlist:
        in_specs.append(pl.BlockSpec(wq_.shape,
                                     (lambda b: (0, 0)) if wq_.ndim == 2
                                     else (lambda b: (0, 0, 0))))
        ins.append(wq_)
    in_specs.append(pl.BlockSpec(wout.shape, lambda b: (0, 0, 0)))
    ins.append(wout)
    in_specs.append(pl.BlockSpec(bout.shape, lambda b: (0, 0)))
    ins.append(bout)

    out_shapes = (
        jax.ShapeDtypeStruct((n, rcores[3], 32), jnp.float32),
        jax.ShapeDtypeStruct((n, rcores[0], 128), jnp.float32),
        jax.ShapeDtypeStruct((n, rcores[1], 128), jnp.float32),
        jax.ShapeDtypeStruct((n, rcores[2], 128), jnp.float32),
    )
    out_specs = tuple(
        pl.BlockSpec((1, s.shape[1], s.shape[2]), lambda b: (b, 0, 0))
        for s in out_shapes)

    scratch = [
        pltpu.VMEM((rexts[2], 128), jnp.float32),
        pltpu.VMEM((rexts[3], 128), jnp.bfloat16),
        pltpu.VMEM((rexts[3], 128), jnp.bfloat16),
        pltpu.VMEM((rcores[3], 128), jnp.float32),
        pltpu.VMEM((rexts[0], 128), jnp.float32),
        pltpu.VMEM((rexts[1], 128), jnp.float32),
        pltpu.VMEM((rexts[2], 128), jnp.float32),
        pltpu.VMEM((rexts[3], 128), jnp.bfloat16),
    ]

    outs = pl.pallas_call(
        _body,
        out_shape=out_shapes,
        grid=(n,),
        in_specs=in_specs,
        out_specs=out_specs,
        scratch_shapes=scratch,
        compiler_params=pltpu.CompilerParams(
            dimension_semantics=("parallel",),
            vmem_limit_bytes=_VLIM,
        ),
    )(*ins)
    o4, y1, y2, y3 = outs
    return (_unpack_out(o4, 64, 8, 4),
            [_unpack_out(y1, 8, 1, 128), _unpack_out(y2, 16, 2, 64),
             _unpack_out(y3, 32, 4, 32)])


def kernel(f0, f1, f2, f3, f4,
           up1_w1x1, up1_b1x1,
           up1_conv_w1, up1_conv_b1,
           up1_conv_bn1_gamma, up1_conv_bn1_beta, up1_conv_bn1_mean, up1_conv_bn1_var,
           up1_conv_w2, up1_conv_b2,
           up1_conv_bn2_gamma, up1_conv_bn2_beta, up1_conv_bn2_mean, up1_conv_bn2_var,
           up2_w1x1, up2_b1x1,
           up2_conv_w1, up2_conv_b1,
           up2_conv_bn1_gamma, up2_conv_bn1_beta, up2_conv_bn1_mean, up2_conv_bn1_var,
           up2_conv_w2, up2_conv_b2,
           up2_conv_bn2_gamma, up2_conv_bn2_beta, up2_conv_bn2_mean, up2_conv_bn2_var,
           up3_w1x1, up3_b1x1,
           up3_conv_w1, up3_conv_b1,
           up3_conv_bn1_gamma, up3_conv_bn1_beta, up3_conv_bn1_mean, up3_conv_bn1_var,
           up3_conv_w2, up3_conv_b2,
           up3_conv_bn2_gamma, up3_conv_bn2_beta, up3_conv_bn2_mean, up3_conv_bn2_var,
           up4_w1x1, up4_b1x1,
           up4_conv_w1, up4_conv_b1,
           up4_conv_bn1_gamma, up4_conv_bn1_beta, up4_conv_bn1_mean, up4_conv_bn1_var,
           up4_conv_w2, up4_conv_b2,
           up4_conv_bn2_gamma, up4_conv_bn2_beta, up4_conv_bn2_mean, up4_conv_bn2_var,
           out_conv_w, out_conv_b):
    def _bn(g, be, m, v):
        return dict(gamma=g, beta=be, mean=m, var=v)

    def _conv(w1, b1, g1, be1, m1, v1, w2, b2, g2, be2, m2, v2):
        return dict(w1=w1, b1=b1, bn1=_bn(g1, be1, m1, v1),
                    w2=w2, b2=b2, bn2=_bn(g2, be2, m2, v2))

    def _up(w1x1, b1x1, *c):
        return dict(w1x1=w1x1, b1x1=b1x1, conv=_conv(*c))

    params = dict(
        up1=_up(up1_w1x1, up1_b1x1,
                up1_conv_w1, up1_conv_b1,
                up1_conv_bn1_gamma, up1_conv_bn1_beta, up1_conv_bn1_mean, up1_conv_bn1_var,
                up1_conv_w2, up1_conv_b2,
                up1_conv_bn2_gamma, up1_conv_bn2_beta, up1_conv_bn2_mean, up1_conv_bn2_var),
        up2=_up(up2_w1x1, up2_b1x1,
                up2_conv_w1, up2_conv_b1,
                up2_conv_bn1_gamma, up2_conv_bn1_beta, up2_conv_bn1_mean, up2_conv_bn1_var,
                up2_conv_w2, up2_conv_b2,
                up2_conv_bn2_gamma, up2_conv_bn2_beta, up2_conv_bn2_mean, up2_conv_bn2_var),
        up3=_up(up3_w1x1, up3_b1x1,
                up3_conv_w1, up3_conv_b1,
                up3_conv_bn1_gamma, up3_conv_bn1_beta, up3_conv_bn1_mean, up3_conv_bn1_var,
                up3_conv_w2, up3_conv_b2,
                up3_conv_bn2_gamma, up3_conv_bn2_beta, up3_conv_bn2_mean, up3_conv_bn2_var),
        up4=_up(up4_w1x1, up4_b1x1,
                up4_conv_w1, up4_conv_b1,
                up4_conv_bn1_gamma, up4_conv_bn1_beta, up4_conv_bn1_mean, up4_conv_bn1_var,
                up4_conv_w2, up4_conv_b2,
                up4_conv_bn2_gamma, up4_conv_bn2_beta, up4_conv_bn2_mean, up4_conv_bn2_var),
        out_conv=dict(w=out_conv_w, b=out_conv_b),
    )
    return _decoder([f0, f1, f2, f3, f4], params)


# speedup vs baseline: 1.0468x; 1.0468x over previous
import math

import numpy as np

import jax
import jax.numpy as jnp
from jax.experimental import pallas as pl
from jax.experimental.pallas import tpu as pltpu

_BN_EPS = 1e-5
_SLOPE = 0.01
_VLIM = 32 * 1024 * 1024
_TAPS9 = [(dy, dw) for dy in range(3) for dw in range(3)]

_LV = [
    (4, 4, 8, 8, 256, 128),
    (8, 8, 16, 16, 128, 64),
    (16, 16, 32, 32, 64, 32),
    (32, 32, 64, 64, 32, 16),
]
_WQ = 8
_WQP = 10
_GQ = 11


def _pk_rows(h):
    rcore = (h + 2) * _WQP
    return rcore, rcore + 2 * _GQ


def _upmat(n_in, n_out):
    u = np.zeros((n_out, n_in), np.float32)
    if n_in == 1:
        u[:, 0] = 1.0
        return u
    for i in range(n_out):
        s = i * (n_in - 1) / (n_out - 1)
        lo = int(math.floor(s))
        hi = min(lo + 1, n_in - 1)
        f = s - lo
        u[i, lo] += 1.0 - f
        u[i, hi] += f
    return u


def _hlerp_taps(h1, h):
    taps = []
    for i in range(h):
        s = i * (h1 - 1) / (h - 1) if h > 1 else 0.0
        lo = int(math.floor(s))
        hi = min(lo + 1, h1 - 1)
        taps.append((lo, hi, s - lo))
    return taps


def _pk_mask(h):
    m = np.zeros((h + 2, _WQP), np.float32)
    m[1:h + 1, 1:_WQ + 1] = 1.0
    return m.reshape(-1, 1)


def _pack_taps(k_hwio, p, n_pad=128):
    k = k_hwio.astype(jnp.float32)
    cin, cout = k.shape[2], k.shape[3]
    m = np.zeros((3, 3, p, p), np.float32)
    for wi, dw in enumerate((-1, 0, 1)):
        for xi, dx in enumerate((-1, 0, 1)):
            for i in range(p):
                for j in range(p):
                    if dw * p + i - j == dx:
                        m[wi, xi, i, j] = 1.0
    t = jnp.einsum('yxco,wxij->ywicjo', k, jnp.asarray(m))
    t = t.reshape(9, p * cin, p * cout)
    if p * cout < n_pad:
        t = jnp.pad(t, ((0, 0), (0, 0), (0, n_pad - p * cout)))
    return t.astype(jnp.bfloat16)


def _tile_row(v, p, n_pad=128):
    v = jnp.tile(v.astype(jnp.float32).reshape(-1), p).reshape(1, -1)
    if v.shape[1] < n_pad:
        v = jnp.pad(v, ((0, 0), (0, n_pad - v.shape[1])))
    return v


def _bn_fold(bias, bn):
    s = bn["gamma"] / jnp.sqrt(bn["var"] + _BN_EPS)
    t = (bias - bn["mean"]) * s + bn["beta"]
    return s, t


def _body(*refs):
    xs = refs[0:5]
    masks = refs[5:9]
    wrefs = refs[9:49]
    wout, bout = refs[49], refs[50]
    o_out, y1o, y2o, y3o = refs[51:55]
    z, up, yt, acc = refs[55:59]
    ybufs = refs[59:63]
    youts = [y1o, y2o, y3o, None]

    for lvl in range(4):
        h1, w1, h, w, c1, c2 = _LV[lvl]
        p2 = 128 // c2
        rcq, rxq = _pk_rows(h)
        if lvl == 0:
            z_wp, z_g, z_rows, z_lanes, slen = 6, 7, 50, 128, w1
        else:
            z_wp, z_g = _WQP, _GQ
            z_rows = _pk_rows(h1)[1]
            z_lanes, slen = 64, _WQ
        (w1pk, b1pk, uw, w1a, w1b, s1, t1, w2c, s2, t2) = \
            wrefs[10 * lvl:10 * lvl + 10]

        if lvl == 0:
            xin = xs[0][0]
        else:
            xin = ybufs[lvl - 1][...].astype(jnp.bfloat16)
        z[pl.ds(0, z_rows), :] = (
            jnp.dot(xin, w1pk[...], preferred_element_type=jnp.float32)
            + b1pk[...])

        p1 = p2 // 2
        up[pl.ds(0, rxq), :] = jnp.zeros((rxq, 128), up.dtype)
        for i, (lo, hi, f) in enumerate(_hlerp_taps(h1, h)):
            zl = z[pl.ds(z_g + (lo + 1) * z_wp + 1, slen), :z_lanes]
            if f == 0.0:
                ap = zl
            else:
                zh = z[pl.ds(z_g + (hi + 1) * z_wp + 1, slen), :z_lanes]
                ap = (1.0 - f) * zl + f * zh
            if lvl == 0:
                au = ap
            else:
                au = jnp.concatenate(
                    [ap[:, k * c2:(k + 1) * c2] for k in range(p1)], axis=0)
            ru = jnp.dot(uw[...], au, preferred_element_type=jnp.float32)
            if p2 == 1:
                rp = ru
            else:
                rp = jnp.concatenate(
                    [ru[j * _WQ:(j + 1) * _WQ, :] for j in range(p2)], axis=1)
            up[pl.ds(_GQ + (i + 1) * _WQP + 1, _WQ), :] = rp.astype(up.dtype)

        x2r = xs[lvl + 1]
        acc[pl.ds(0, rcq), :] = jnp.zeros((rcq, 128), jnp.float32)
        for t, (dy, dw) in enumerate(_TAPS9):
            off = _GQ + (dy - 1) * _WQP + (dw - 1)
            acc[pl.ds(0, rcq), :] += jnp.dot(
                x2r[0, pl.ds(off, rcq), :], w1a[t],
                preferred_element_type=jnp.float32)
            acc[pl.ds(0, rcq), :] += jnp.dot(
                up[pl.ds(off, rcq), :], w1b[t],
                preferred_element_type=jnp.float32)
        y1v = acc[pl.ds(0, rcq), :] * s1[...] + t1[...]
        y1v = jnp.where(y1v >= 0.0, y1v, _SLOPE * y1v) * masks[lvl][...]
        yt[pl.ds(0, _GQ), :] = jnp.zeros((_GQ, 128), yt.dtype)
        yt[pl.ds(_GQ, rcq), :] = y1v.astype(yt.dtype)
        yt[pl.ds(_GQ + rcq, _GQ), :] = jnp.zeros((_GQ, 128), yt.dtype)

        acc[pl.ds(0, rcq), :] = jnp.zeros((rcq, 128), jnp.float32)
        for t, (dy, dw) in enumerate(_TAPS9):
            off = _GQ + (dy - 1) * _WQP + (dw - 1)
            acc[pl.ds(0, rcq), :] += jnp.dot(
                yt[pl.ds(off, rcq), :], w2c[t],
                preferred_element_type=jnp.float32)
        y2v = acc[pl.ds(0, rcq), :] * s2[...] + t2[...]
        y2v = jnp.where(y2v >= 0.0, y2v, _SLOPE * y2v) * masks[lvl][...]

        yb = ybufs[lvl]
        yb[pl.ds(0, _GQ), :] = jnp.zeros((_GQ, 128), yb.dtype)
        yb[pl.ds(_GQ, rcq), :] = y2v.astype(yb.dtype)
        yb[pl.ds(_GQ + rcq, _GQ), :] = jnp.zeros((_GQ, 128), yb.dtype)
        if lvl < 3:
            youts[lvl][0] = y2v

    rcq = _pk_rows(64)[0]
    acc[pl.ds(0, rcq), :] = jnp.zeros((rcq, 128), jnp.float32)
    for t, (dy, dw) in enumerate(_TAPS9):
        off = _GQ + (dy - 1) * _WQP + (dw - 1)
        acc[pl.ds(0, rcq), :] += jnp.dot(
            ybufs[3][pl.ds(off, rcq), :], wout[t],
            preferred_element_type=jnp.float32)
    o_out[0] = acc[pl.ds(0, rcq), :32] + bout[:, :32]


def _pack_ext(x_nchw, p):
    n, c, h, w = x_nchw.shape
    wq = (w * c) // 128
    x = jnp.transpose(x_nchw, (0, 2, 3, 1)).reshape(n, h, wq, 128)
    xp = jnp.pad(x, ((0, 0), (1, 1), (1, 1), (0, 0)))
    flat = xp.reshape(n, (h + 2) * (wq + 2), 128)
    return jnp.pad(flat, ((0, 0), (_GQ, _GQ), (0, 0))).astype(jnp.bfloat16)


def _ext_unpacked(x_nchw):
    n, c, h, w = x_nchw.shape
    x = jnp.transpose(x_nchw, (0, 2, 3, 1))
    xp = jnp.pad(x, ((0, 0), (1, 1), (1, 1), (0, 0)))
    flat = xp.reshape(n, (h + 2) * (w + 2), c)
    g = w + 3
    return jnp.pad(flat, ((0, 0), (g, g), (0, 0))).astype(jnp.bfloat16)


def _unpack_out(a, h, p, c):
    n = a.shape[0]
    b = a.reshape(n, h + 2, _WQP, a.shape[-1])[:, 1:h + 1, 1:_WQ + 1, :p * c]
    b = b.reshape(n, h, _WQ * p, c)
    return jnp.transpose(b, (0, 3, 1, 2))


def _decoder(feats_nchw, params):
    n = feats_nchw[0].shape[0]
    exts = [_ext_unpacked(feats_nchw[4])]
    for lvl in range(4):
        c2 = _LV[lvl][5]
        exts.append(_pack_ext(feats_nchw[3 - lvl], 128 // c2))
    masks = [jnp.asarray(_pk_mask(lv[2])) for lv in _LV]

    wlist = []
    for lvl, name in enumerate(("up1", "up2", "up3", "up4")):
        p = params[name]
        h1, w1, h, w, c1, c2 = _LV[lvl]
        p2 = 128 // c2
        p1 = p2 // 2
        w1x1 = p["w1x1"].astype(jnp.float32)
        if lvl == 0:
            w1pk = w1x1.astype(jnp.bfloat16)
            b1pk = _tile_row(p["b1x1"], 1)
        else:
            w1pk = jnp.einsum('co,ij->icjo', w1x1, jnp.eye(p1, dtype=jnp.float32))
            w1pk = w1pk.reshape(p1 * c1, p1 * c2)
            w1pk = jnp.pad(w1pk, ((0, 0), (0, 128 - p1 * c2))).astype(jnp.bfloat16)
            b1pk = _tile_row(p["b1x1"], p1)
        u_np = _upmat(w1, w)
        if lvl > 0:
            rmap = np.array([(r % _WQ) * p2 + r // _WQ for r in range(w)])
            cmap = np.array([(c % _WQ) * p1 + c // _WQ for c in range(w1)])
            u_np = u_np[np.ix_(rmap, cmap)]
        uw = jnp.asarray(u_np)
        wc1 = p["conv"]["w1"]
        w1a = _pack_taps(wc1[:, :, :c2, :], p2)
        w1b = _pack_taps(wc1[:, :, c2:, :], p2)
        w2c = _pack_taps(p["conv"]["w2"], p2)
        s1, t1 = _bn_fold(p["conv"]["b1"], p["conv"]["bn1"])
        s2, t2 = _bn_fold(p["conv"]["b2"], p["conv"]["bn2"])
        wlist += [w1pk, b1pk, uw, w1a, w1b, _tile_row(s1, p2), _tile_row(t1, p2),
                  w2c, _tile_row(s2, p2), _tile_row(t2, p2)]

    wout = _pack_taps(params["out_conv"]["w"], 8)
    bout = _tile_row(params["out_conv"]["b"], 8)

    rcores = [_pk_rows(lv[2])[0] for lv in _LV]
    rexts = [_pk_rows(lv[2])[1] for lv in _LV]

    in_specs = []
    ins = []
    for e in exts:
        in_specs.append(pl.BlockSpec((1, e.shape[1], e.shape[2]),
                                     lambda b: (b, 0, 0)))
        ins.append(e)
    for m in masks:
        in_specs.append(pl.BlockSpec(m.shape, lambda b: (0, 0)))
        ins.append(m)
    for wq_ in wlist:
        in_specs.append(pl.BlockSpec(wq_.shape,
                                     (lambda b: (0, 0)) if wq_.ndim == 2
                                     else (lambda b: (0, 0, 0))))
        ins.append(wq_)
    in_specs.append(pl.BlockSpec(wout.shape, lambda b: (0, 0, 0)))
    ins.append(wout)
    in_specs.append(pl.BlockSpec(bout.shape, lambda b: (0, 0)))
    ins.append(bout)

    out_shapes = (
        jax.ShapeDtypeStruct((n, rcores[3], 32), jnp.float32),
        jax.ShapeDtypeStruct((n, rcores[0], 128), jnp.float32),
        jax.ShapeDtypeStruct((n, rcores[1], 128), jnp.float32),
        jax.ShapeDtypeStruct((n, rcores[2], 128), jnp.float32),
    )
    out_specs = tuple(
        pl.BlockSpec((1, s.shape[1], s.shape[2]), lambda b: (b, 0, 0))
        for s in out_shapes)

    scratch = [
        pltpu.VMEM((rexts[2], 128), jnp.float32),
        pltpu.VMEM((rexts[3], 128), jnp.bfloat16),
        pltpu.VMEM((rexts[3], 128), jnp.bfloat16),
        pltpu.VMEM((rcores[3], 128), jnp.float32),
        pltpu.VMEM((rexts[0], 128), jnp.float32),
        pltpu.VMEM((rexts[1], 128), jnp.float32),
        pltpu.VMEM((rexts[2], 128), jnp.float32),
        pltpu.VMEM((rexts[3], 128), jnp.bfloat16),
    ]

    outs = pl.pallas_call(
        _body,
        out_shape=out_shapes,
        grid=(n,),
        in_specs=in_specs,
        out_specs=out_specs,
        scratch_shapes=scratch,
        compiler_params=pltpu.CompilerParams(
            dimension_semantics=("parallel",),
            vmem_limit_bytes=_VLIM,
        ),
    )(*ins)
    o4, y1, y2, y3 = outs
    return (_unpack_out(o4, 64, 8, 4),
            [_unpack_out(y1, 8, 1, 128), _unpack_out(y2, 16, 2, 64),
             _unpack_out(y3, 32, 4, 32)])


def kernel(f0, f1, f2, f3, f4,
           up1_w1x1, up1_b1x1,
           up1_conv_w1, up1_conv_b1,
           up1_conv_bn1_gamma, up1_conv_bn1_beta, up1_conv_bn1_mean, up1_conv_bn1_var,
           up1_conv_w2, up1_conv_b2,
           up1_conv_bn2_gamma, up1_conv_bn2_beta, up1_conv_bn2_mean, up1_conv_bn2_var,
           up2_w1x1, up2_b1x1,
           up2_conv_w1, up2_conv_b1,
           up2_conv_bn1_gamma, up2_conv_bn1_beta, up2_conv_bn1_mean, up2_conv_bn1_var,
           up2_conv_w2, up2_conv_b2,
           up2_conv_bn2_gamma, up2_conv_bn2_beta, up2_conv_bn2_mean, up2_conv_bn2_var,
           up3_w1x1, up3_b1x1,
           up3_conv_w1, up3_conv_b1,
           up3_conv_bn1_gamma, up3_conv_bn1_beta, up3_conv_bn1_mean, up3_conv_bn1_var,
           up3_conv_w2, up3_conv_b2,
           up3_conv_bn2_gamma, up3_conv_bn2_beta, up3_conv_bn2_mean, up3_conv_bn2_var,
           up4_w1x1, up4_b1x1,
           up4_conv_w1, up4_conv_b1,
           up4_conv_bn1_gamma, up4_conv_bn1_beta, up4_conv_bn1_mean, up4_conv_bn1_var,
           up4_conv_w2, up4_conv_b2,
           up4_conv_bn2_gamma, up4_conv_bn2_beta, up4_conv_bn2_mean, up4_conv_bn2_var,
           out_conv_w, out_conv_b):
    def _bn(g, be, m, v):
        return dict(gamma=g, beta=be, mean=m, var=v)

    def _conv(w1, b1, g1, be1, m1, v1, w2, b2, g2, be2, m2, v2):
        return dict(w1=w1, b1=b1, bn1=_bn(g1, be1, m1, v1),
                    w2=w2, b2=b2, bn2=_bn(g2, be2, m2, v2))

    def _up(w1x1, b1x1, *c):
        return dict(w1x1=w1x1, b1x1=b1x1, conv=_conv(*c))

    params = dict(
        up1=_up(up1_w1x1, up1_b1x1,
                up1_conv_w1, up1_conv_b1,
                up1_conv_bn1_gamma, up1_conv_bn1_beta, up1_conv_bn1_mean, up1_conv_bn1_var,
                up1_conv_w2, up1_conv_b2,
                up1_conv_bn2_gamma, up1_conv_bn2_beta, up1_conv_bn2_mean, up1_conv_bn2_var),
        up2=_up(up2_w1x1, up2_b1x1,
                up2_conv_w1, up2_conv_b1,
                up2_conv_bn1_gamma, up2_conv_bn1_beta, up2_conv_bn1_mean, up2_conv_bn1_var,
                up2_conv_w2, up2_conv_b2,
                up2_conv_bn2_gamma, up2_conv_bn2_beta, up2_conv_bn2_mean, up2_conv_bn2_var),
        up3=_up(up3_w1x1, up3_b1x1,
                up3_conv_w1, up3_conv_b1,
                up3_conv_bn1_gamma, up3_conv_bn1_beta, up3_conv_bn1_mean, up3_conv_bn1_var,
                up3_conv_w2, up3_conv_b2,
                up3_conv_bn2_gamma, up3_conv_bn2_beta, up3_conv_bn2_mean, up3_conv_bn2_var),
        up4=_up(up4_w1x1, up4_b1x1,
                up4_conv_w1, up4_conv_b1,
                up4_conv_bn1_gamma, up4_conv_bn1_beta, up4_conv_bn1_mean, up4_conv_bn1_var,
                up4_conv_w2, up4_conv_b2,
                up4_conv_bn2_gamma, up4_conv_bn2_beta, up4_conv_bn2_mean, up4_conv_bn2_var),
        out_conv=dict(w=out_conv_w, b=out_conv_b),
    )
    return _decoder([f0, f1, f2, f3, f4], params)


# speedup vs baseline: 1.0998x; 1.0506x over previous
import math

import numpy as np

import jax
import jax.numpy as jnp
from jax.experimental import pallas as pl
from jax.experimental.pallas import tpu as pltpu

_BN_EPS = 1e-5
_SLOPE = 0.01
_VLIM = 32 * 1024 * 1024
_TAPS9 = [(dy, dw) for dy in range(3) for dw in range(3)]

_LV = [
    (4, 4, 8, 8, 256, 128),
    (8, 8, 16, 16, 128, 64),
    (16, 16, 32, 32, 64, 32),
    (32, 32, 64, 64, 32, 16),
]
_WQ = 8
_WQP = 10
_GQ = 11


def _pk_rows(h):
    rcore = (h + 2) * _WQP
    return rcore, rcore + 2 * _GQ


def _upmat(n_in, n_out):
    u = np.zeros((n_out, n_in), np.float32)
    if n_in == 1:
        u[:, 0] = 1.0
        return u
    for i in range(n_out):
        s = i * (n_in - 1) / (n_out - 1)
        lo = int(math.floor(s))
        hi = min(lo + 1, n_in - 1)
        f = s - lo
        u[i, lo] += 1.0 - f
        u[i, hi] += f
    return u


def _hlerp_taps(h1, h):
    taps = []
    for i in range(h):
        s = i * (h1 - 1) / (h - 1) if h > 1 else 0.0
        lo = int(math.floor(s))
        hi = min(lo + 1, h1 - 1)
        taps.append((lo, hi, s - lo))
    return taps


def _pk_mask(h):
    m = np.zeros((h + 2, _WQP), np.float32)
    m[1:h + 1, 1:_WQ + 1] = 1.0
    return m.reshape(-1, 1)[_WQP:_WQP + h * _WQP]


def _pack_taps(k_hwio, p, n_pad=128):
    k = k_hwio.astype(jnp.float32)
    cin, cout = k.shape[2], k.shape[3]
    m = np.zeros((3, 3, p, p), np.float32)
    for wi, dw in enumerate((-1, 0, 1)):
        for xi, dx in enumerate((-1, 0, 1)):
            for i in range(p):
                for j in range(p):
                    if dw * p + i - j == dx:
                        m[wi, xi, i, j] = 1.0
    t = jnp.einsum('yxco,wxij->ywicjo', k, jnp.asarray(m))
    t = t.reshape(9, p * cin, p * cout)
    if p * cout < n_pad:
        t = jnp.pad(t, ((0, 0), (0, 0), (0, n_pad - p * cout)))
    return t.astype(jnp.bfloat16)


def _tile_row(v, p, n_pad=128):
    v = jnp.tile(v.astype(jnp.float32).reshape(-1), p).reshape(1, -1)
    if v.shape[1] < n_pad:
        v = jnp.pad(v, ((0, 0), (0, n_pad - v.shape[1])))
    return v


def _bn_fold(bias, bn):
    s = bn["gamma"] / jnp.sqrt(bn["var"] + _BN_EPS)
    t = (bias - bn["mean"]) * s + bn["beta"]
    return s, t


def _body(*refs):
    xs = refs[0:5]
    masks = refs[5:9]
    wrefs = refs[9:49]
    wout, bout = refs[49], refs[50]
    o_out, y1o, y2o, y3o = refs[51:55]
    z, up, yt, acc = refs[55:59]
    ybufs = refs[59:63]
    youts = [y1o, y2o, y3o, None]

    for lvl in range(4):
        h1, w1, h, w, c1, c2 = _LV[lvl]
        p2 = 128 // c2
        rcq, rxq = _pk_rows(h)
        if lvl == 0:
            z_wp, z_g, z_rows, z_lanes, slen = 6, 7, 50, 128, w1
        else:
            z_wp, z_g = _WQP, _GQ
            z_rows = _pk_rows(h1)[1]
            z_lanes, slen = 64, _WQ
        (w1pk, b1pk, uw, w1a, w1b, s1, t1, w2c, s2, t2) = \
            wrefs[10 * lvl:10 * lvl + 10]

        if lvl == 0:
            xin = xs[0][0]
        else:
            xin = ybufs[lvl - 1][...].astype(jnp.bfloat16)
        z[pl.ds(0, z_rows), :] = (
            jnp.dot(xin, w1pk[...], preferred_element_type=jnp.float32)
            + b1pk[...])

        p1 = p2 // 2
        up[pl.ds(0, rxq), :] = jnp.zeros((rxq, 128), up.dtype)
        for i, (lo, hi, f) in enumerate(_hlerp_taps(h1, h)):
            zl = z[pl.ds(z_g + (lo + 1) * z_wp + 1, slen), :z_lanes]
            if f == 0.0:
                ap = zl
            else:
                zh = z[pl.ds(z_g + (hi + 1) * z_wp + 1, slen), :z_lanes]
                ap = (1.0 - f) * zl + f * zh
            if lvl == 0:
                au = ap
            else:
                au = jnp.concatenate(
                    [ap[:, k * c2:(k + 1) * c2] for k in range(p1)], axis=0)
            ru = jnp.dot(uw[...], au, preferred_element_type=jnp.float32)
            if p2 == 1:
                rp = ru
            else:
                rp = jnp.concatenate(
                    [ru[j * _WQ:(j + 1) * _WQ, :] for j in range(p2)], axis=1)
            up[pl.ds(_GQ + (i + 1) * _WQP + 1, _WQ), :] = rp.astype(up.dtype)

        x2r = xs[lvl + 1]
        rci = h * _WQP
        head = _GQ + _WQP
        tail = rxq - head - rci
        acc[pl.ds(0, rci), :] = jnp.zeros((rci, 128), jnp.float32)
        for t, (dy, dw) in enumerate(_TAPS9):
            off = head + (dy - 1) * _WQP + (dw - 1)
            acc[pl.ds(0, rci), :] += jnp.dot(
                x2r[0, pl.ds(off, rci), :], w1a[t],
                preferred_element_type=jnp.float32)
            acc[pl.ds(0, rci), :] += jnp.dot(
                up[pl.ds(off, rci), :], w1b[t],
                preferred_element_type=jnp.float32)
        y1v = acc[pl.ds(0, rci), :] * s1[...] + t1[...]
        y1v = jnp.where(y1v >= 0.0, y1v, _SLOPE * y1v) * masks[lvl][...]
        yt[pl.ds(0, head), :] = jnp.zeros((head, 128), yt.dtype)
        yt[pl.ds(head, rci), :] = y1v.astype(yt.dtype)
        yt[pl.ds(head + rci, tail), :] = jnp.zeros((tail, 128), yt.dtype)

        acc[pl.ds(0, rci), :] = jnp.zeros((rci, 128), jnp.float32)
        for t, (dy, dw) in enumerate(_TAPS9):
            off = head + (dy - 1) * _WQP + (dw - 1)
            acc[pl.ds(0, rci), :] += jnp.dot(
                yt[pl.ds(off, rci), :], w2c[t],
                preferred_element_type=jnp.float32)
        y2v = acc[pl.ds(0, rci), :] * s2[...] + t2[...]
        y2v = jnp.where(y2v >= 0.0, y2v, _SLOPE * y2v) * masks[lvl][...]

        yb = ybufs[lvl]
        yb[pl.ds(0, head), :] = jnp.zeros((head, 128), yb.dtype)
        yb[pl.ds(head, rci), :] = y2v.astype(yb.dtype)
        yb[pl.ds(head + rci, tail), :] = jnp.zeros((tail, 128), yb.dtype)
        if lvl < 3:
            youts[lvl][0] = y2v

    rci = 64 * _WQP
    head = _GQ + _WQP
    acc[pl.ds(0, rci), :] = jnp.zeros((rci, 128), jnp.float32)
    for t, (dy, dw) in enumerate(_TAPS9):
        off = head + (dy - 1) * _WQP + (dw - 1)
        acc[pl.ds(0, rci), :] += jnp.dot(
            ybufs[3][pl.ds(off, rci), :], wout[t],
            preferred_element_type=jnp.float32)
    o_out[0] = acc[pl.ds(0, rci), :32] + bout[:, :32]


def _pack_ext(x_nchw, p):
    n, c, h, w = x_nchw.shape
    wq = (w * c) // 128
    x = jnp.transpose(x_nchw, (0, 2, 3, 1)).reshape(n, h, wq, 128)
    xp = jnp.pad(x, ((0, 0), (1, 1), (1, 1), (0, 0)))
    flat = xp.reshape(n, (h + 2) * (wq + 2), 128)
    return jnp.pad(flat, ((0, 0), (_GQ, _GQ), (0, 0))).astype(jnp.bfloat16)


def _ext_unpacked(x_nchw):
    n, c, h, w = x_nchw.shape
    x = jnp.transpose(x_nchw, (0, 2, 3, 1))
    xp = jnp.pad(x, ((0, 0), (1, 1), (1, 1), (0, 0)))
    flat = xp.reshape(n, (h + 2) * (w + 2), c)
    g = w + 3
    return jnp.pad(flat, ((0, 0), (g, g), (0, 0))).astype(jnp.bfloat16)


def _unpack_out(a, h, p, c):
    n = a.shape[0]
    b = a.reshape(n, h, _WQP, a.shape[-1])[:, :, 1:_WQ + 1, :p * c]
    b = b.reshape(n, h, _WQ * p, c)
    return jnp.transpose(b, (0, 3, 1, 2))


def _decoder(feats_nchw, params):
    n = feats_nchw[0].shape[0]
    exts = [_ext_unpacked(feats_nchw[4])]
    for lvl in range(4):
        c2 = _LV[lvl][5]
        exts.append(_pack_ext(feats_nchw[3 - lvl], 128 // c2))
    masks = [jnp.asarray(_pk_mask(lv[2])) for lv in _LV]

    wlist = []
    for lvl, name in enumerate(("up1", "up2", "up3", "up4")):
        p = params[name]
        h1, w1, h, w, c1, c2 = _LV[lvl]
        p2 = 128 // c2
        p1 = p2 // 2
        w1x1 = p["w1x1"].astype(jnp.float32)
        if lvl == 0:
            w1pk = w1x1.astype(jnp.bfloat16)
            b1pk = _tile_row(p["b1x1"], 1)
        else:
            w1pk = jnp.einsum('co,ij->icjo', w1x1, jnp.eye(p1, dtype=jnp.float32))
            w1pk = w1pk.reshape(p1 * c1, p1 * c2)
            w1pk = jnp.pad(w1pk, ((0, 0), (0, 128 - p1 * c2))).astype(jnp.bfloat16)
            b1pk = _tile_row(p["b1x1"], p1)
        u_np = _upmat(w1, w)
        if lvl > 0:
            rmap = np.array([(r % _WQ) * p2 + r // _WQ for r in range(w)])
            cmap = np.array([(c % _WQ) * p1 + c // _WQ for c in range(w1)])
            u_np = u_np[np.ix_(rmap, cmap)]
        uw = jnp.asarray(u_np)
        wc1 = p["conv"]["w1"]
        w1a = _pack_taps(wc1[:, :, :c2, :], p2)
        w1b = _pack_taps(wc1[:, :, c2:, :], p2)
        w2c = _pack_taps(p["conv"]["w2"], p2)
        s1, t1 = _bn_fold(p["conv"]["b1"], p["conv"]["bn1"])
        s2, t2 = _bn_fold(p["conv"]["b2"], p["conv"]["bn2"])
        wlist += [w1pk, b1pk, uw, w1a, w1b, _tile_row(s1, p2), _tile_row(t1, p2),
                  w2c, _tile_row(s2, p2), _tile_row(t2, p2)]

    wout = _pack_taps(params["out_conv"]["w"], 8)
    bout = _tile_row(params["out_conv"]["b"], 8)

    rcores = [_pk_rows(lv[2])[0] for lv in _LV]
    rexts = [_pk_rows(lv[2])[1] for lv in _LV]

    in_specs = []
    ins = []
    for e in exts:
        in_specs.append(pl.BlockSpec((1, e.shape[1], e.shape[2]),
                                     lambda b: (b, 0, 0)))
        ins.append(e)
    for m in masks:
        in_specs.append(pl.BlockSpec(m.shape, lambda b: (0, 0)))
        ins.append(m)
    for wq_ in wlist:
        in_specs.append(pl.BlockSpec(wq_.shape,
                                     (lambda b: (0, 0)) if wq_.ndim == 2
                                     else (lambda b: (0, 0, 0))))
        ins.append(wq_)
    in_specs.append(pl.BlockSpec(wout.shape, lambda b: (0, 0, 0)))
    ins.append(wout)
    in_specs.append(pl.BlockSpec(bout.shape, lambda b: (0, 0)))
    ins.append(bout)

    rints = [lv[2] * _WQP for lv in _LV]
    out_shapes = (
        jax.ShapeDtypeStruct((n, rints[3], 32), jnp.float32),
        jax.ShapeDtypeStruct((n, rints[0], 128), jnp.float32),
        jax.ShapeDtypeStruct((n, rints[1], 128), jnp.float32),
        jax.ShapeDtypeStruct((n, rints[2], 128), jnp.float32),
    )
    out_specs = tuple(
        pl.BlockSpec((1, s.shape[1], s.shape[2]), lambda b: (b, 0, 0))
        for s in out_shapes)

    scratch = [
        pltpu.VMEM((rexts[2], 128), jnp.float32),
        pltpu.VMEM((rexts[3], 128), jnp.bfloat16),
        pltpu.VMEM((rexts[3], 128), jnp.bfloat16),
        pltpu.VMEM((rcores[3], 128), jnp.float32),
        pltpu.VMEM((rexts[0], 128), jnp.float32),
        pltpu.VMEM((rexts[1], 128), jnp.float32),
        pltpu.VMEM((rexts[2], 128), jnp.float32),
        pltpu.VMEM((rexts[3], 128), jnp.bfloat16),
    ]

    outs = pl.pallas_call(
        _body,
        out_shape=out_shapes,
        grid=(n,),
        in_specs=in_specs,
        out_specs=out_specs,
        scratch_shapes=scratch,
        compiler_params=pltpu.CompilerParams(
            dimension_semantics=("parallel",),
            vmem_limit_bytes=_VLIM,
        ),
    )(*ins)
    o4, y1, y2, y3 = outs
    return (_unpack_out(o4, 64, 8, 4),
            [_unpack_out(y1, 8, 1, 128), _unpack_out(y2, 16, 2, 64),
             _unpack_out(y3, 32, 4, 32)])


def kernel(f0, f1, f2, f3, f4,
           up1_w1x1, up1_b1x1,
           up1_conv_w1, up1_conv_b1,
           up1_conv_bn1_gamma, up1_conv_bn1_beta, up1_conv_bn1_mean, up1_conv_bn1_var,
           up1_conv_w2, up1_conv_b2,
           up1_conv_bn2_gamma, up1_conv_bn2_beta, up1_conv_bn2_mean, up1_conv_bn2_var,
           up2_w1x1, up2_b1x1,
           up2_conv_w1, up2_conv_b1,
           up2_conv_bn1_gamma, up2_conv_bn1_beta, up2_conv_bn1_mean, up2_conv_bn1_var,
           up2_conv_w2, up2_conv_b2,
           up2_conv_bn2_gamma, up2_conv_bn2_beta, up2_conv_bn2_mean, up2_conv_bn2_var,
           up3_w1x1, up3_b1x1,
           up3_conv_w1, up3_conv_b1,
           up3_conv_bn1_gamma, up3_conv_bn1_beta, up3_conv_bn1_mean, up3_conv_bn1_var,
           up3_conv_w2, up3_conv_b2,
           up3_conv_bn2_gamma, up3_conv_bn2_beta, up3_conv_bn2_mean, up3_conv_bn2_var,
           up4_w1x1, up4_b1x1,
           up4_conv_w1, up4_conv_b1,
           up4_conv_bn1_gamma, up4_conv_bn1_beta, up4_conv_bn1_mean, up4_conv_bn1_var,
           up4_conv_w2, up4_conv_b2,
           up4_conv_bn2_gamma, up4_conv_bn2_beta, up4_conv_bn2_mean, up4_conv_bn2_var,
           out_conv_w, out_conv_b):
    def _bn(g, be, m, v):
        return dict(gamma=g, beta=be, mean=m, var=v)

    def _conv(w1, b1, g1, be1, m1, v1, w2, b2, g2, be2, m2, v2):
        return dict(w1=w1, b1=b1, bn1=_bn(g1, be1, m1, v1),
                    w2=w2, b2=b2, bn2=_bn(g2, be2, m2, v2))

    def _up(w1x1, b1x1, *c):
        return dict(w1x1=w1x1, b1x1=b1x1, conv=_conv(*c))

    params = dict(
        up1=_up(up1_w1x1, up1_b1x1,
                up1_conv_w1, up1_conv_b1,
                up1_conv_bn1_gamma, up1_conv_bn1_beta, up1_conv_bn1_mean, up1_conv_bn1_var,
                up1_conv_w2, up1_conv_b2,
                up1_conv_bn2_gamma, up1_conv_bn2_beta, up1_conv_bn2_mean, up1_conv_bn2_var),
        up2=_up(up2_w1x1, up2_b1x1,
                up2_conv_w1, up2_conv_b1,
                up2_conv_bn1_gamma, up2_conv_bn1_beta, up2_conv_bn1_mean, up2_conv_bn1_var,
                up2_conv_w2, up2_conv_b2,
                up2_conv_bn2_gamma, up2_conv_bn2_beta, up2_conv_bn2_mean, up2_conv_bn2_var),
        up3=_up(up3_w1x1, up3_b1x1,
                up3_conv_w1, up3_conv_b1,
                up3_conv_bn1_gamma, up3_conv_bn1_beta, up3_conv_bn1_mean, up3_conv_bn1_var,
                up3_conv_w2, up3_conv_b2,
                up3_conv_bn2_gamma, up3_conv_bn2_beta, up3_conv_bn2_mean, up3_conv_bn2_var),
        up4=_up(up4_w1x1, up4_b1x1,
                up4_conv_w1, up4_conv_b1,
                up4_conv_bn1_gamma, up4_conv_bn1_beta, up4_conv_bn1_mean, up4_conv_bn1_var,
                up4_conv_w2, up4_conv_b2,
                up4_conv_bn2_gamma, up4_conv_bn2_beta, up4_conv_bn2_mean, up4_conv_bn2_var),
        out_conv=dict(w=out_conv_w, b=out_conv_b),
    )
    return _decoder([f0, f1, f2, f3, f4], params)
